# Initial kernel scaffold; baseline (speedup 1.0000x reference)
import jax
import jax.numpy as jnp
from jax.experimental import pallas as pl


def kernel(x, edge_index, edge_attr, batch, node_W, node_b, edge_W, edge_b, c1_W1, c1_b1, c1_W2, c1_b2, c2_W1, c2_b1, c2_W2, c2_b2, m_W1, m_b1, m_W2, m_b2):
    raise NotImplementedError("write your pallas kernel here")


# speedup vs baseline: 1.7284x; 1.7284x over previous
import functools

import jax
import jax.numpy as jnp
from jax import lax
from jax.experimental import pallas as pl
from jax.experimental.pallas import tpu as pltpu
from jax.experimental.pallas import tpu_sc as plsc

_H = 128
_N = 10000
_E = 320000
_G = 64
_CH = 80
_SIZES = [10080] * 11 + [9840] * 4 + [9760]
_HALF = 160000
_MAXCH = _SIZES[0] // _CH
_EPAD = _E + 640


def _lane(vec, i):
    return lax.squeeze(lax.slice(vec, (i,), (i + 1,)), (0,))


def _sc_layer(table, s_src, s_ea, s_dst, vvec, bvec):
    n = table.shape[0]
    mesh = plsc.VectorSubcoreMesh(core_axis_name="c", subcore_axis_name="s")
    nblk = (n + 16) // 16

    scratch = [
        pltpu.VMEM((_CH,), jnp.int32),
        pltpu.VMEM((_CH,), jnp.float32),
        pltpu.VMEM((_CH + 16,), jnp.int32),
        pltpu.VMEM((_CH, _H), jnp.float32),
        pltpu.VMEM((_H,), jnp.float32),
        pltpu.VMEM((_H,), jnp.float32),
        pltpu.VMEM((16, _H), jnp.float32),
        pltpu.VMEM_SHARED((n + 16, _H), jnp.float32),
        pltpu.SemaphoreType.DMA,
    ]

    @functools.partial(
        pl.kernel, mesh=mesh,
        out_type=jax.ShapeDtypeStruct((2, n, _H), jnp.float32),
        scratch_types=scratch)
    def body(tab_hbm, src_hbm, ea_hbm, dst_hbm, v_hbm, b_hbm, out_hbm,
             src_v, ea_v, dst_v, rows_v, v_v, b_v, zbuf, acc, sem):
        cid = lax.axis_index("c")
        sid = lax.axis_index("s")

        pltpu.sync_copy(v_hbm, v_v)
        pltpu.sync_copy(b_hbm, b_v)
        vr = [v_v[pl.ds(16 * j, 16)] for j in range(8)]
        br = [b_v[pl.ds(16 * j, 16)] for j in range(8)]

        iota = lax.iota(jnp.int32, 16)
        start = cid * _HALF + jnp.where(sid <= 11, 10080 * sid,
                                        110880 + 9840 * (sid - 11))
        size = jnp.where(sid < 11, 10080, jnp.where(sid < 15, 9840, 9760))
        end = start + size

        z16 = jnp.zeros((16,), jnp.float32)
        for i in range(16):
            for j in range(8):
                zbuf[i, pl.ds(16 * j, 16)] = z16

        nround = (nblk + 15) // 16

        def zcopy(k, c):
            m = sid + 16 * k
            mm = jnp.minimum(m, nblk - 1)
            offc = pl.multiple_of(mm * 16, 16)
            pltpu.sync_copy(zbuf, acc.at[pl.ds(offc, 16)])
            return c

        lax.fori_loop(0, nround, zcopy, 0)
        plsc.subcore_barrier()

        zrow = [z16 for _ in range(8)]

        def chunk(t, carry):
            racc = list(carry[0])
            prev = carry[1]
            base = pl.multiple_of(start + t * _CH, 8)
            pltpu.sync_copy(src_hbm.at[pl.ds(base, _CH)], src_v)
            pltpu.sync_copy(ea_hbm.at[pl.ds(base, _CH)], ea_v)
            pltpu.sync_copy(dst_hbm.at[pl.ds(base, _CH + 16)], dst_v)
            pltpu.async_copy(tab_hbm.at[src_v], rows_v, sem).wait()

            def grp(g, carry2):
                racc2 = list(carry2[0])
                prev2 = carry2[1]
                eav = ea_v[pl.ds(g * 16, 16)]
                dv = dst_v[pl.ds(g * 16, 16)]
                dnext = dst_v[pl.ds((g + 1) * 16, 16)]
                dov = iota + n
                for i in range(16):
                    pos = start + t * _CH + g * 16 + i
                    d = _lane(dv, i)
                    dn = _lane(dnext, 0) if i == 15 else _lane(dv, i + 1)
                    b_ea = _lane(eav, i)
                    active = pos < end
                    same = (d == prev2) & active
                    flush = ((d != dn) | (pos == end - 1)) & active
                    samef = jnp.where(same, 1.0, 0.0)
                    r = g * 16 + i
                    for j in range(8):
                        ej = b_ea * vr[j] + br[j]
                        m = jnp.maximum(rows_v[r, pl.ds(16 * j, 16)] + ej, 0.0)
                        m = jnp.where(active, m, 0.0)
                        racc2[j] = m + racc2[j] * samef
                        rows_v[r, pl.ds(16 * j, 16)] = racc2[j]
                    dsel = jnp.where(flush, d, n + i)
                    dov = jnp.where(iota == i, dsel, dov)
                    prev2 = d
                pltpu.sync_copy(rows_v.at[pl.ds(g * 16, 16)], acc.at[dov],
                                add=True)
                return (tuple(racc2), prev2)

            return lax.fori_loop(0, _CH // 16, grp, (tuple(racc), prev))

        lax.fori_loop(0, _MAXCH, chunk, (tuple(zrow), jnp.int32(-1)))

        plsc.subcore_barrier()

        nout = n // 16

        def ocopy(k, c):
            m = sid + 16 * k
            mm = jnp.minimum(m, nout - 1)
            off = pl.multiple_of(mm * 16, 16)
            pltpu.sync_copy(acc.at[pl.ds(off, 16)],
                            out_hbm.at[cid, pl.ds(off, 16)])
            return c

        lax.fori_loop(0, (nout + 15) // 16, ocopy, 0)

    return body(table, s_src, s_ea, s_dst, vvec, bvec)


def _tc_encode(x, node_w, node_b):
    n = x.shape[0]
    bn = 1000

    def body(x_ref, nw_ref, nb_ref, o_ref):
        o_ref[...] = x_ref[...] * nw_ref[...] + nb_ref[...]

    return pl.pallas_call(
        body,
        grid=(n // bn,),
        in_specs=[pl.BlockSpec((bn, 1), lambda i: (i, 0)),
                  pl.BlockSpec((1, _H), lambda i: (0, 0)),
                  pl.BlockSpec((1, _H), lambda i: (0, 0))],
        out_specs=pl.BlockSpec((bn, _H), lambda i: (i, 0)),
        out_shape=jax.ShapeDtypeStruct((n, _H), jnp.float32),
    )(x, node_w, node_b.reshape(1, _H))


def _tc_mlp(h, agg_a, agg_b, w1, b1, w2, b2):
    n = h.shape[0]
    bn = 1000

    def body(h_ref, aa_ref, ab_ref, w1_ref, b1_ref, w2_ref, b2_ref, o_ref):
        z = h_ref[...] + (aa_ref[...] + ab_ref[...])
        y = jnp.dot(z, w1_ref[...], preferred_element_type=jnp.float32) + b1_ref[...]
        y = jnp.where(y >= 0, y, 0.01 * y)
        o = jnp.dot(y, w2_ref[...], preferred_element_type=jnp.float32) + b2_ref[...]
        o_ref[...] = jnp.maximum(o, 0.0)

    return pl.pallas_call(
        body,
        grid=(n // bn,),
        in_specs=[pl.BlockSpec((bn, _H), lambda i: (i, 0)),
                  pl.BlockSpec((bn, _H), lambda i: (i, 0)),
                  pl.BlockSpec((bn, _H), lambda i: (i, 0)),
                  pl.BlockSpec((_H, _H), lambda i: (0, 0)),
                  pl.BlockSpec((1, _H), lambda i: (0, 0)),
                  pl.BlockSpec((_H, _H), lambda i: (0, 0)),
                  pl.BlockSpec((1, _H), lambda i: (0, 0))],
        out_specs=pl.BlockSpec((bn, _H), lambda i: (i, 0)),
        out_shape=jax.ShapeDtypeStruct((n, _H), jnp.float32),
    )(h, agg_a, agg_b, w1, b1.reshape(1, _H), w2, b2.reshape(1, _H))


def _tc_pool_head(h, batch2d, batch_row, m_w1, m_b1, m_w2, m_b2s):
    n = h.shape[0]

    def body(h_ref, b2_ref, br_ref, w1_ref, b1_ref, w2_ref, bb_ref, o_ref, acc):
        acc[...] = jnp.zeros((_G, _H), jnp.float32)

        def fold(i, c):
            b = b2_ref[i, 0]
            acc[pl.ds(b, 1), :] = acc[pl.ds(b, 1), :] + h_ref[pl.ds(i, 1), :]
            return c

        lax.fori_loop(0, n, fold, 0)
        rows = lax.broadcasted_iota(jnp.int32, (_G, n), 0)
        oh = (rows == br_ref[...]).astype(jnp.float32)
        counts = jnp.sum(oh, axis=1, keepdims=True)
        g = acc[...] / jnp.maximum(counts, 1.0)
        o = jnp.dot(g, w1_ref[...], preferred_element_type=jnp.float32) + b1_ref[...]
        o = jnp.where(o >= 0, o, 0.01 * o)
        logits = jnp.dot(o, w2_ref[...], preferred_element_type=jnp.float32) + bb_ref[...]
        o_ref[...] = jax.nn.sigmoid(logits)

    return pl.pallas_call(
        body,
        out_shape=jax.ShapeDtypeStruct((_G, 1), jnp.float32),
        scratch_shapes=[pltpu.VMEM((_G, _H), jnp.float32)],
    )(h, batch2d, batch_row, m_w1, m_b1.reshape(1, _H), m_w2, m_b2s)


def kernel(x, edge_index, edge_attr, batch, node_W, node_b, edge_W, edge_b,
           c1_W1, c1_b1, c1_W2, c1_b2, c2_W1, c2_b1, c2_W2, c2_b2,
           m_W1, m_b1, m_W2, m_b2):
    src = edge_index[0]
    dst = edge_index[1]
    ea = edge_attr[:, 0]
    v = edge_W[0]

    perm = jnp.argsort(dst, stable=True)
    pad = _EPAD - _E
    s_src = jnp.pad(src[perm], (0, pad))
    s_ea = jnp.pad(ea[perm], (0, pad))
    s_dst = jnp.pad(dst[perm], (0, pad))

    h0 = _tc_encode(x, node_W, node_b)
    agg1 = _sc_layer(h0, s_src, s_ea, s_dst, v, edge_b)
    h1 = _tc_mlp(h0, agg1[0], agg1[1], c1_W1, c1_b1, c1_W2, c1_b2)
    agg2 = _sc_layer(h1, s_src, s_ea, s_dst, v, edge_b)
    h2 = _tc_mlp(h1, agg2[0], agg2[1], c2_W1, c2_b1, c2_W2, c2_b2)
    out = _tc_pool_head(h2, batch[:, None], batch[None, :], m_W1, m_b1,
                        m_W2, m_b2[None, :])
    return out[:, 0]
